# baseline (speedup 1.0000x reference)
import functools

import jax
import jax.numpy as jnp
from jax import lax
from jax.experimental import pallas as pl
from jax.experimental.pallas import tpu as pltpu
from jax.experimental.pallas import tpu_sc as plsc

_B = 256
_C = 100
_V = 1000
_D = 64
_HC = 64
_N = 50
_E = 800
_E2 = 2 * _E
_P = 37

_NP = _B // 2
_PB = 8
_NSTEP = _NP // _PB
_PL = 2 * _D

_NW = 32
_ROWS = _B * _C
_RPW = _ROWS // _NW
_CHUNK = 80
_NCHUNK = _RPW // _CHUNK


def _dot(a, b):
    return lax.dot(a, b, preferred_element_type=jnp.float32)


def _gather_sc(table, idx3):
    mesh = plsc.VectorSubcoreMesh(core_axis_name="c", subcore_axis_name="s")

    @functools.partial(
        pl.kernel,
        mesh=mesh,
        out_type=jax.ShapeDtypeStruct((_ROWS, _D), jnp.float32),
        scratch_types=[
            pltpu.VMEM((_NCHUNK, _CHUNK), jnp.int32),
            pltpu.VMEM((_RPW, _D), jnp.float32),
            pltpu.SemaphoreType.DMA,
        ],
        compiler_params=pltpu.CompilerParams(use_tc_tiling_on_sc=False),
    )
    def gk(table_hbm, idx_hbm, out_hbm, idx_v, rows_v, sem):
        wid = lax.axis_index("s") * 2 + lax.axis_index("c")
        pltpu.sync_copy(idx_hbm.at[wid], idx_v)
        copies = [
            pltpu.async_copy(
                table_hbm.at[idx_v.at[j]],
                rows_v.at[pl.ds(j * _CHUNK, _CHUNK)],
                sem,
            )
            for j in range(_NCHUNK)
        ]
        for cp in copies:
            cp.wait()
        pltpu.sync_copy(rows_v, out_hbm.at[pl.ds(wid * _RPW, _RPW)])

    return gk(table, idx3)


def _bd2(m):
    za = jnp.zeros_like(m)
    top = jnp.concatenate([m, za], axis=1)
    bot = jnp.concatenate([za, m], axis=1)
    return jnp.concatenate([top, bot], axis=0)


def _tc_body(x2_ref, ei2_ref, ei2t_ref, wz_ref, wh_ref, lzw_ref, lhw_ref,
             bz_ref, bh_ref, lzb_ref, lhb_ref, att_ref, lospair_ref,
             cw1_ref, cb1_ref, cw2_ref, cb2_ref, out_ref,
             a2_ref, bdzh_ref, czh_ref, sad_ref, sdis_ref):
    f32 = jnp.float32
    pid = pl.program_id(0)

    @pl.when(pid == 0)
    def _build():
        dst_row = ei2_ref[1:2, :]
        src_col = ei2t_ref[:, 0:1]
        dst_col = ei2t_ref[:, 1:2]
        io_ne = lax.broadcasted_iota(jnp.int32, (_C, _E2), 0)
        io_en = lax.broadcasted_iota(jnp.int32, (_E2, _C), 1)
        dst1ht = jnp.where(io_ne == dst_row, 1.0, 0.0).astype(f32)
        src1h = jnp.where(io_en == src_col, 1.0, 0.0).astype(f32)
        dst1h = jnp.where(io_en == dst_col, 1.0, 0.0).astype(f32)
        eye = jnp.where(
            lax.broadcasted_iota(jnp.int32, (_C, _C), 0)
            == lax.broadcasted_iota(jnp.int32, (_C, _C), 1),
            1.0, 0.0).astype(f32)
        acount = _dot(dst1ht, src1h) + eye
        deg_col = _dot(dst1ht, jnp.ones((_E2, 1), f32)) + 1.0
        deg_row = _dot(jnp.ones((1, _E2), f32), dst1h) + 1.0
        dinv_col = jnp.where(deg_col > 0, lax.rsqrt(deg_col), 0.0)
        dinv_row = jnp.where(deg_row > 0, lax.rsqrt(deg_row), 0.0)
        a2_ref[...] = acount * dinv_col * dinv_row

        lz1 = lzw_ref[0:_HC, :]
        lh1 = lhw_ref[0:_HC, :]
        bdzh_ref[...] = jnp.concatenate(
            [_bd2(_dot(wz_ref[...], lz1)), _bd2(_dot(wh_ref[...], lh1))],
            axis=1)
        c_z = _dot(bz_ref[...], lz1) + lzb_ref[...]
        c_h = _dot(bh_ref[...], lh1) + lhb_ref[...]
        czh_ref[...] = jnp.concatenate([c_z, c_z, c_h, c_h], axis=1)

    mask_ad = jnp.where(
        lax.broadcasted_iota(jnp.int32, (_C, 1), 0) < _N, 1.0, 0.0).astype(f32)
    y_all = _dot(a2_ref[...], x2_ref[...])
    czh = czh_ref[...]
    srows_ad, srows_dis = [], []
    for i in range(_PB):
        y = y_all[:, i * _PL:(i + 1) * _PL]
        zh = _dot(y, bdzh_ref[...]) + czh
        z = 0.5 * (1.0 + jnp.tanh(0.5 * zh[:, 0:_PL]))
        t = jnp.tanh(zh[:, _PL:2 * _PL])
        hn = (1.0 - z) * t
        srows_ad.append(jnp.sum(hn * mask_ad, axis=0, keepdims=True))
        srows_dis.append(jnp.sum(hn * (1.0 - mask_ad), axis=0, keepdims=True))
    sad_ref[pl.ds(pid * _PB, _PB), :] = jnp.concatenate(srows_ad, axis=0)
    sdis_ref[pl.ds(pid * _PB, _PB), :] = jnp.concatenate(srows_dis, axis=0)

    @pl.when(pid == _NSTEP - 1)
    def _finish():
        att = att_ref[...]
        ex = jnp.exp(att - jnp.max(att, axis=1, keepdims=True))
        probs = ex / jnp.sum(ex, axis=1, keepdims=True)
        io_p = lax.broadcasted_iota(jnp.int32, (_NP, _P), 1)

        def wcols(los_col):
            wa = jnp.sum(jnp.where(io_p < los_col, probs, 0.0),
                         axis=1, keepdims=True)
            wd = jnp.sum(jnp.where(io_p >= los_col, probs, 0.0),
                         axis=1, keepdims=True)
            return (jnp.broadcast_to(wa, (_NP, _HC)),
                    jnp.broadcast_to(wd, (_NP, _HC)))

        wa_e, wd_e = wcols(lospair_ref[:, 0:1])
        wa_o, wd_o = wcols(lospair_ref[:, 1:2])
        w_ad = jnp.concatenate([wa_e, wa_o], axis=1)
        w_dis = jnp.concatenate([wd_e, wd_o], axis=1)
        pooled = (w_ad * sad_ref[...] + w_dis * sdis_ref[...]) * (1.0 / _N)
        cb1 = cb1_ref[...]
        cb2 = cb2_ref[...]
        h = jnp.maximum(_dot(pooled, _bd2(cw1_ref[...]))
                        + jnp.concatenate([cb1, cb1], axis=1), 0.0)
        out_ref[...] = (_dot(h, _bd2(cw2_ref[...]))
                        + jnp.concatenate([cb2, cb2], axis=1))


def _tc_forward(x2, lospair, ei2, ei2t, wz, wh, lzw, lhw,
                bz2, bh2, lzb2, lhb2, att2, cw1, cb1, cw2, cb2):
    def rep(shape):
        return pl.BlockSpec(shape, lambda i: (0,) * len(shape))

    in_specs = [
        pl.BlockSpec((_C, _PB * _PL), lambda i: (0, i)),
        rep((2, _E2)), rep((_E2, 2)),
        rep((_D, _HC)), rep((_D, _HC)),
        rep((2 * _HC, _HC)), rep((2 * _HC, _HC)),
        rep((1, _HC)), rep((1, _HC)), rep((1, _HC)), rep((1, _HC)),
        rep((1, _P)), rep((_NP, 2)),
        rep((_HC, 2 * _HC)), rep((1, 2 * _HC)), rep((2 * _HC, 1)), rep((1, 1)),
    ]
    return pl.pallas_call(
        _tc_body,
        grid=(_NSTEP,),
        in_specs=in_specs,
        out_specs=pl.BlockSpec((_NP, 2), lambda i: (0, 0)),
        out_shape=jax.ShapeDtypeStruct((_NP, 2), jnp.float32),
        scratch_shapes=[
            pltpu.VMEM((_C, _C), jnp.float32),
            pltpu.VMEM((_PL, 2 * _PL), jnp.float32),
            pltpu.VMEM((1, 2 * _PL), jnp.float32),
            pltpu.VMEM((_NP, _PL), jnp.float32),
            pltpu.VMEM((_NP, _PL), jnp.float32),
        ],
    )(x2, ei2, ei2t, wz, wh, lzw, lhw,
      bz2, bh2, lzb2, lhb2, att2, lospair,
      cw1, cb1, cw2, cb2)


def kernel(x_batch, LOS_batch, template_edge_index, emb, W_z, b_z, W_r, b_r,
           W_h, b_h, lz_W, lz_b, lr_W, lr_b, lh_W, lh_b, attention,
           cls_W1, cls_b1, cls_W2, cls_b2):
    del W_r, b_r, lr_W, lr_b
    table = emb.reshape(_C * _V, _D)
    offs = (jnp.arange(_C, dtype=jnp.int32) * _V)[:, None]
    idx3 = (x_batch.astype(jnp.int32).T + offs).reshape(_NW, _NCHUNK, _CHUNK)
    g = _gather_sc(table, idx3)
    ei = template_edge_index.astype(jnp.int32)
    ei2 = jnp.concatenate([ei, ei + _N], axis=1)
    out2 = _tc_forward(
        g.reshape(_C, _B * _D),
        LOS_batch.astype(jnp.int32).reshape(_NP, 2),
        ei2, ei2.T,
        W_z, W_h, lz_W, lh_W,
        b_z.reshape(1, _HC), b_h.reshape(1, _HC),
        lz_b.reshape(1, _HC), lh_b.reshape(1, _HC),
        attention.reshape(1, _P),
        cls_W1, cls_b1.reshape(1, 2 * _HC), cls_W2, cls_b2.reshape(1, 1))
    return out2.reshape(_B, 1)


# speedup vs baseline: 1.0459x; 1.0459x over previous
import functools

import jax
import jax.numpy as jnp
from jax import lax
from jax.experimental import pallas as pl
from jax.experimental.pallas import tpu as pltpu
from jax.experimental.pallas import tpu_sc as plsc

_B = 256
_C = 100
_V = 1000
_D = 64
_HC = 64
_N = 50
_E = 800
_E2 = 2 * _E
_P = 37

_NP = _B // 2
_PB = 16
_NSTEP = _NP // _PB
_PL = 2 * _D

_NW = 32
_ROWS = _B * _C
_RPW = _ROWS // _NW
_CHUNK = 80
_NCHUNK = _RPW // _CHUNK


def _dot(a, b):
    return lax.dot(a, b, preferred_element_type=jnp.float32)


def _gather_sc(table, idx3):
    mesh = plsc.VectorSubcoreMesh(core_axis_name="c", subcore_axis_name="s")

    @functools.partial(
        pl.kernel,
        mesh=mesh,
        out_type=jax.ShapeDtypeStruct((_ROWS, _D), jnp.float32),
        scratch_types=[
            pltpu.VMEM((_NCHUNK, _CHUNK), jnp.int32),
            pltpu.VMEM((_RPW, _D), jnp.float32),
            pltpu.SemaphoreType.DMA,
        ],
        compiler_params=pltpu.CompilerParams(use_tc_tiling_on_sc=False),
    )
    def gk(table_hbm, idx_hbm, out_hbm, idx_v, rows_v, sem):
        wid = lax.axis_index("s") * 2 + lax.axis_index("c")
        pltpu.sync_copy(idx_hbm.at[wid], idx_v)
        copies = [
            pltpu.async_copy(
                table_hbm.at[idx_v.at[j]],
                rows_v.at[pl.ds(j * _CHUNK, _CHUNK)],
                sem,
            )
            for j in range(_NCHUNK)
        ]
        for cp in copies:
            cp.wait()
        pltpu.sync_copy(rows_v, out_hbm.at[pl.ds(wid * _RPW, _RPW)])

    return gk(table, idx3)


def _bd2(m):
    za = jnp.zeros_like(m)
    top = jnp.concatenate([m, za], axis=1)
    bot = jnp.concatenate([za, m], axis=1)
    return jnp.concatenate([top, bot], axis=0)


def _tc_body(x2_ref, ei2_ref, ei2t_ref, wz_ref, wh_ref, lzw_ref, lhw_ref,
             bz_ref, bh_ref, lzb_ref, lhb_ref, att_ref, lospair_ref,
             cw1_ref, cb1_ref, cw2_ref, cb2_ref, out_ref,
             a2_ref, bdzh_ref, czh_ref, sad_ref, sdis_ref):
    f32 = jnp.float32
    pid = pl.program_id(0)

    @pl.when(pid == 0)
    def _build():
        dst_row = ei2_ref[1:2, :]
        src_col = ei2t_ref[:, 0:1]
        dst_col = ei2t_ref[:, 1:2]
        io_ne = lax.broadcasted_iota(jnp.int32, (_C, _E2), 0)
        io_en = lax.broadcasted_iota(jnp.int32, (_E2, _C), 1)
        dst1ht = jnp.where(io_ne == dst_row, 1.0, 0.0).astype(f32)
        src1h = jnp.where(io_en == src_col, 1.0, 0.0).astype(f32)
        dst1h = jnp.where(io_en == dst_col, 1.0, 0.0).astype(f32)
        eye = jnp.where(
            lax.broadcasted_iota(jnp.int32, (_C, _C), 0)
            == lax.broadcasted_iota(jnp.int32, (_C, _C), 1),
            1.0, 0.0).astype(f32)
        acount = _dot(dst1ht, src1h) + eye
        deg_col = _dot(dst1ht, jnp.ones((_E2, 1), f32)) + 1.0
        deg_row = _dot(jnp.ones((1, _E2), f32), dst1h) + 1.0
        dinv_col = jnp.where(deg_col > 0, lax.rsqrt(deg_col), 0.0)
        dinv_row = jnp.where(deg_row > 0, lax.rsqrt(deg_row), 0.0)
        a2_ref[...] = acount * dinv_col * dinv_row

        lz1 = lzw_ref[0:_HC, :]
        lh1 = lhw_ref[0:_HC, :]
        bdzh_ref[...] = jnp.concatenate(
            [_bd2(_dot(wz_ref[...], lz1)), _bd2(_dot(wh_ref[...], lh1))],
            axis=1)
        c_z = _dot(bz_ref[...], lz1) + lzb_ref[...]
        c_h = _dot(bh_ref[...], lh1) + lhb_ref[...]
        czh_ref[...] = jnp.concatenate([c_z, c_z, c_h, c_h], axis=1)

    mask_ad = jnp.where(
        lax.broadcasted_iota(jnp.int32, (_C, 1), 0) < _N, 1.0, 0.0).astype(f32)
    y_all = _dot(a2_ref[...], x2_ref[...])
    czh = czh_ref[...]
    srows_ad, srows_dis = [], []
    for i in range(_PB):
        y = y_all[:, i * _PL:(i + 1) * _PL]
        zh = _dot(y, bdzh_ref[...]) + czh
        z = 0.5 * (1.0 + jnp.tanh(0.5 * zh[:, 0:_PL]))
        t = jnp.tanh(zh[:, _PL:2 * _PL])
        hn = (1.0 - z) * t
        srows_ad.append(jnp.sum(hn * mask_ad, axis=0, keepdims=True))
        srows_dis.append(jnp.sum(hn * (1.0 - mask_ad), axis=0, keepdims=True))
    sad_ref[pl.ds(pid * _PB, _PB), :] = jnp.concatenate(srows_ad, axis=0)
    sdis_ref[pl.ds(pid * _PB, _PB), :] = jnp.concatenate(srows_dis, axis=0)

    @pl.when(pid == _NSTEP - 1)
    def _finish():
        att = att_ref[...]
        ex = jnp.exp(att - jnp.max(att, axis=1, keepdims=True))
        probs = ex / jnp.sum(ex, axis=1, keepdims=True)
        io_p = lax.broadcasted_iota(jnp.int32, (_NP, _P), 1)

        def wcols(los_col):
            wa = jnp.sum(jnp.where(io_p < los_col, probs, 0.0),
                         axis=1, keepdims=True)
            wd = jnp.sum(jnp.where(io_p >= los_col, probs, 0.0),
                         axis=1, keepdims=True)
            return (jnp.broadcast_to(wa, (_NP, _HC)),
                    jnp.broadcast_to(wd, (_NP, _HC)))

        wa_e, wd_e = wcols(lospair_ref[:, 0:1])
        wa_o, wd_o = wcols(lospair_ref[:, 1:2])
        w_ad = jnp.concatenate([wa_e, wa_o], axis=1)
        w_dis = jnp.concatenate([wd_e, wd_o], axis=1)
        pooled = (w_ad * sad_ref[...] + w_dis * sdis_ref[...]) * (1.0 / _N)
        cb1 = cb1_ref[...]
        cb2 = cb2_ref[...]
        h = jnp.maximum(_dot(pooled, _bd2(cw1_ref[...]))
                        + jnp.concatenate([cb1, cb1], axis=1), 0.0)
        out_ref[...] = (_dot(h, _bd2(cw2_ref[...]))
                        + jnp.concatenate([cb2, cb2], axis=1))


def _tc_forward(x2, lospair, ei2, ei2t, wz, wh, lzw, lhw,
                bz2, bh2, lzb2, lhb2, att2, cw1, cb1, cw2, cb2):
    def rep(shape):
        return pl.BlockSpec(shape, lambda i: (0,) * len(shape))

    in_specs = [
        pl.BlockSpec((_C, _PB * _PL), lambda i: (0, i)),
        rep((2, _E2)), rep((_E2, 2)),
        rep((_D, _HC)), rep((_D, _HC)),
        rep((2 * _HC, _HC)), rep((2 * _HC, _HC)),
        rep((1, _HC)), rep((1, _HC)), rep((1, _HC)), rep((1, _HC)),
        rep((1, _P)), rep((_NP, 2)),
        rep((_HC, 2 * _HC)), rep((1, 2 * _HC)), rep((2 * _HC, 1)), rep((1, 1)),
    ]
    return pl.pallas_call(
        _tc_body,
        grid=(_NSTEP,),
        in_specs=in_specs,
        out_specs=pl.BlockSpec((_NP, 2), lambda i: (0, 0)),
        out_shape=jax.ShapeDtypeStruct((_NP, 2), jnp.float32),
        scratch_shapes=[
            pltpu.VMEM((_C, _C), jnp.float32),
            pltpu.VMEM((_PL, 2 * _PL), jnp.float32),
            pltpu.VMEM((1, 2 * _PL), jnp.float32),
            pltpu.VMEM((_NP, _PL), jnp.float32),
            pltpu.VMEM((_NP, _PL), jnp.float32),
        ],
    )(x2, ei2, ei2t, wz, wh, lzw, lhw,
      bz2, bh2, lzb2, lhb2, att2, lospair,
      cw1, cb1, cw2, cb2)


def kernel(x_batch, LOS_batch, template_edge_index, emb, W_z, b_z, W_r, b_r,
           W_h, b_h, lz_W, lz_b, lr_W, lr_b, lh_W, lh_b, attention,
           cls_W1, cls_b1, cls_W2, cls_b2):
    del W_r, b_r, lr_W, lr_b
    table = emb.reshape(_C * _V, _D)
    offs = (jnp.arange(_C, dtype=jnp.int32) * _V)[:, None]
    idx3 = (x_batch.astype(jnp.int32).T + offs).reshape(_NW, _NCHUNK, _CHUNK)
    g = _gather_sc(table, idx3)
    ei = template_edge_index.astype(jnp.int32)
    ei2 = jnp.concatenate([ei, ei + _N], axis=1)
    out2 = _tc_forward(
        g.reshape(_C, _B * _D),
        LOS_batch.astype(jnp.int32).reshape(_NP, 2),
        ei2, ei2.T,
        W_z, W_h, lz_W, lh_W,
        b_z.reshape(1, _HC), b_h.reshape(1, _HC),
        lz_b.reshape(1, _HC), lh_b.reshape(1, _HC),
        attention.reshape(1, _P),
        cls_W1, cls_b1.reshape(1, 2 * _HC), cls_W2, cls_b2.reshape(1, 1))
    return out2.reshape(_B, 1)


# speedup vs baseline: 1.0629x; 1.0162x over previous
import functools

import jax
import jax.numpy as jnp
from jax import lax
from jax.experimental import pallas as pl
from jax.experimental.pallas import tpu as pltpu
from jax.experimental.pallas import tpu_sc as plsc

_B = 256
_C = 100
_V = 1000
_D = 64
_HC = 64
_N = 50
_E = 800
_E2 = 2 * _E
_P = 37

_NP = _B // 2
_PB = 32
_NSTEP = _NP // _PB
_PL = 2 * _D

_NW = 32
_ROWS = _B * _C
_RPW = _ROWS // _NW
_CHUNK = 80
_NCHUNK = _RPW // _CHUNK


def _dot(a, b):
    return lax.dot(a, b, preferred_element_type=jnp.float32)


def _gather_sc(table, idx3):
    mesh = plsc.VectorSubcoreMesh(core_axis_name="c", subcore_axis_name="s")

    @functools.partial(
        pl.kernel,
        mesh=mesh,
        out_type=jax.ShapeDtypeStruct((_ROWS, _D), jnp.float32),
        scratch_types=[
            pltpu.VMEM((_NCHUNK, _CHUNK), jnp.int32),
            pltpu.VMEM((_RPW, _D), jnp.float32),
            pltpu.SemaphoreType.DMA,
        ],
        compiler_params=pltpu.CompilerParams(use_tc_tiling_on_sc=False),
    )
    def gk(table_hbm, idx_hbm, out_hbm, idx_v, rows_v, sem):
        wid = lax.axis_index("s") * 2 + lax.axis_index("c")
        pltpu.sync_copy(idx_hbm.at[wid], idx_v)
        copies = [
            pltpu.async_copy(
                table_hbm.at[idx_v.at[j]],
                rows_v.at[pl.ds(j * _CHUNK, _CHUNK)],
                sem,
            )
            for j in range(_NCHUNK)
        ]
        for cp in copies:
            cp.wait()
        pltpu.sync_copy(rows_v, out_hbm.at[pl.ds(wid * _RPW, _RPW)])

    return gk(table, idx3)


def _bd2(m):
    za = jnp.zeros_like(m)
    top = jnp.concatenate([m, za], axis=1)
    bot = jnp.concatenate([za, m], axis=1)
    return jnp.concatenate([top, bot], axis=0)


def _tc_body(x2_ref, ei2_ref, ei2t_ref, wz_ref, wh_ref, lzw_ref, lhw_ref,
             bz_ref, bh_ref, lzb_ref, lhb_ref, att_ref, lospair_ref,
             cw1_ref, cb1_ref, cw2_ref, cb2_ref, out_ref,
             a2_ref, bdzh_ref, czh_ref, sad_ref, sdis_ref):
    f32 = jnp.float32
    pid = pl.program_id(0)

    @pl.when(pid == 0)
    def _build():
        dst_row = ei2_ref[1:2, :]
        src_col = ei2t_ref[:, 0:1]
        dst_col = ei2t_ref[:, 1:2]
        io_ne = lax.broadcasted_iota(jnp.int32, (_C, _E2), 0)
        io_en = lax.broadcasted_iota(jnp.int32, (_E2, _C), 1)
        dst1ht = jnp.where(io_ne == dst_row, 1.0, 0.0).astype(f32)
        src1h = jnp.where(io_en == src_col, 1.0, 0.0).astype(f32)
        dst1h = jnp.where(io_en == dst_col, 1.0, 0.0).astype(f32)
        eye = jnp.where(
            lax.broadcasted_iota(jnp.int32, (_C, _C), 0)
            == lax.broadcasted_iota(jnp.int32, (_C, _C), 1),
            1.0, 0.0).astype(f32)
        acount = _dot(dst1ht, src1h) + eye
        deg_col = _dot(dst1ht, jnp.ones((_E2, 1), f32)) + 1.0
        deg_row = _dot(jnp.ones((1, _E2), f32), dst1h) + 1.0
        dinv_col = jnp.where(deg_col > 0, lax.rsqrt(deg_col), 0.0)
        dinv_row = jnp.where(deg_row > 0, lax.rsqrt(deg_row), 0.0)
        a2_ref[...] = acount * dinv_col * dinv_row

        lz1 = lzw_ref[0:_HC, :]
        lh1 = lhw_ref[0:_HC, :]
        bdzh_ref[...] = jnp.concatenate(
            [_bd2(_dot(wz_ref[...], lz1)), _bd2(_dot(wh_ref[...], lh1))],
            axis=1)
        c_z = _dot(bz_ref[...], lz1) + lzb_ref[...]
        c_h = _dot(bh_ref[...], lh1) + lhb_ref[...]
        czh_ref[...] = jnp.concatenate([c_z, c_z, c_h, c_h], axis=1)

    mask_ad = jnp.where(
        lax.broadcasted_iota(jnp.int32, (_C, 1), 0) < _N, 1.0, 0.0).astype(f32)
    y_all = _dot(a2_ref[...], x2_ref[...])
    czh = czh_ref[...]
    srows_ad, srows_dis = [], []
    for i in range(_PB):
        y = y_all[:, i * _PL:(i + 1) * _PL]
        zh = _dot(y, bdzh_ref[...]) + czh
        z = 0.5 * (1.0 + jnp.tanh(0.5 * zh[:, 0:_PL]))
        t = jnp.tanh(zh[:, _PL:2 * _PL])
        hn = (1.0 - z) * t
        srows_ad.append(jnp.sum(hn * mask_ad, axis=0, keepdims=True))
        srows_dis.append(jnp.sum(hn * (1.0 - mask_ad), axis=0, keepdims=True))
    sad_ref[pl.ds(pid * _PB, _PB), :] = jnp.concatenate(srows_ad, axis=0)
    sdis_ref[pl.ds(pid * _PB, _PB), :] = jnp.concatenate(srows_dis, axis=0)

    @pl.when(pid == _NSTEP - 1)
    def _finish():
        att = att_ref[...]
        ex = jnp.exp(att - jnp.max(att, axis=1, keepdims=True))
        probs = ex / jnp.sum(ex, axis=1, keepdims=True)
        io_p = lax.broadcasted_iota(jnp.int32, (_NP, _P), 1)

        def wcols(los_col):
            wa = jnp.sum(jnp.where(io_p < los_col, probs, 0.0),
                         axis=1, keepdims=True)
            wd = jnp.sum(jnp.where(io_p >= los_col, probs, 0.0),
                         axis=1, keepdims=True)
            return (jnp.broadcast_to(wa, (_NP, _HC)),
                    jnp.broadcast_to(wd, (_NP, _HC)))

        wa_e, wd_e = wcols(lospair_ref[:, 0:1])
        wa_o, wd_o = wcols(lospair_ref[:, 1:2])
        w_ad = jnp.concatenate([wa_e, wa_o], axis=1)
        w_dis = jnp.concatenate([wd_e, wd_o], axis=1)
        pooled = (w_ad * sad_ref[...] + w_dis * sdis_ref[...]) * (1.0 / _N)
        cb1 = cb1_ref[...]
        cb2 = cb2_ref[...]
        h = jnp.maximum(_dot(pooled, _bd2(cw1_ref[...]))
                        + jnp.concatenate([cb1, cb1], axis=1), 0.0)
        out_ref[...] = (_dot(h, _bd2(cw2_ref[...]))
                        + jnp.concatenate([cb2, cb2], axis=1))


def _tc_forward(x2, lospair, ei2, ei2t, wz, wh, lzw, lhw,
                bz2, bh2, lzb2, lhb2, att2, cw1, cb1, cw2, cb2):
    def rep(shape):
        return pl.BlockSpec(shape, lambda i: (0,) * len(shape))

    in_specs = [
        pl.BlockSpec((_C, _PB * _PL), lambda i: (0, i)),
        rep((2, _E2)), rep((_E2, 2)),
        rep((_D, _HC)), rep((_D, _HC)),
        rep((2 * _HC, _HC)), rep((2 * _HC, _HC)),
        rep((1, _HC)), rep((1, _HC)), rep((1, _HC)), rep((1, _HC)),
        rep((1, _P)), rep((_NP, 2)),
        rep((_HC, 2 * _HC)), rep((1, 2 * _HC)), rep((2 * _HC, 1)), rep((1, 1)),
    ]
    return pl.pallas_call(
        _tc_body,
        grid=(_NSTEP,),
        in_specs=in_specs,
        out_specs=pl.BlockSpec((_NP, 2), lambda i: (0, 0)),
        out_shape=jax.ShapeDtypeStruct((_NP, 2), jnp.float32),
        scratch_shapes=[
            pltpu.VMEM((_C, _C), jnp.float32),
            pltpu.VMEM((_PL, 2 * _PL), jnp.float32),
            pltpu.VMEM((1, 2 * _PL), jnp.float32),
            pltpu.VMEM((_NP, _PL), jnp.float32),
            pltpu.VMEM((_NP, _PL), jnp.float32),
        ],
    )(x2, ei2, ei2t, wz, wh, lzw, lhw,
      bz2, bh2, lzb2, lhb2, att2, lospair,
      cw1, cb1, cw2, cb2)


def kernel(x_batch, LOS_batch, template_edge_index, emb, W_z, b_z, W_r, b_r,
           W_h, b_h, lz_W, lz_b, lr_W, lr_b, lh_W, lh_b, attention,
           cls_W1, cls_b1, cls_W2, cls_b2):
    del W_r, b_r, lr_W, lr_b
    table = emb.reshape(_C * _V, _D)
    offs = (jnp.arange(_C, dtype=jnp.int32) * _V)[:, None]
    idx3 = (x_batch.astype(jnp.int32).T + offs).reshape(_NW, _NCHUNK, _CHUNK)
    g = _gather_sc(table, idx3)
    ei = template_edge_index.astype(jnp.int32)
    ei2 = jnp.concatenate([ei, ei + _N], axis=1)
    out2 = _tc_forward(
        g.reshape(_C, _B * _D),
        LOS_batch.astype(jnp.int32).reshape(_NP, 2),
        ei2, ei2.T,
        W_z, W_h, lz_W, lh_W,
        b_z.reshape(1, _HC), b_h.reshape(1, _HC),
        lz_b.reshape(1, _HC), lh_b.reshape(1, _HC),
        attention.reshape(1, _P),
        cls_W1, cls_b1.reshape(1, 2 * _HC), cls_W2, cls_b2.reshape(1, 1))
    return out2.reshape(_B, 1)
